# baseline (speedup 1.0000x reference)
import functools
import math

import jax
import jax.numpy as jnp
import numpy as np
from jax import lax
from jax.experimental import pallas as pl
from jax.experimental.pallas import tpu as pltpu
from jax.experimental.pallas import tpu_sc as plsc

D_MODEL = 1024
B = 4
S = 2048
NUM_MOD = 3

NC = 2
NS = 16
NW = NC * NS
S_PER_W = S // NW
ROWS = 16
N_CHUNKS = S_PER_W // ROWS
CH_PER_M = B * N_CHUNKS
LANES = 16
NBUF = 3


def _pe_np(d_model=D_MODEL, max_len=S):
    position = np.arange(max_len, dtype=np.float32)[:, None]
    div_term = np.exp(
        np.arange(0, d_model, 2, dtype=np.float32) * (-math.log(10000.0) / d_model))
    pe = np.zeros((max_len, d_model), dtype=np.float32)
    pe[:, 0::2] = np.sin(position * div_term)
    pe[:, 1::2] = np.cos(position * div_term)
    return pe


def _pe_bf16_const():
    import ml_dtypes
    return jnp.asarray(_pe_np().astype(ml_dtypes.bfloat16))


def _pe_packed_const():
    import ml_dtypes
    pe = _pe_np().reshape(S, D_MODEL // 32, 32)
    bits = pe.astype(ml_dtypes.bfloat16).view(np.uint16).astype(np.uint32)
    lo, hi = bits[:, :, :16], bits[:, :, 16:]
    words = (lo | (hi << 16)).astype(np.uint32).view(np.int32)
    return jnp.asarray(words.reshape(S, D_MODEL // 2))


_MESH = plsc.VectorSubcoreMesh(core_axis_name="c", subcore_axis_name="s")


@functools.partial(
    pl.kernel,
    mesh=_MESH,
    out_type=jax.ShapeDtypeStruct((B, S, D_MODEL), jnp.float32),
    compiler_params=pltpu.CompilerParams(
        use_tc_tiling_on_sc=True, needs_layout_passes=False,
        skip_device_barrier=True),
    scratch_types=(
        [pltpu.VMEM((S_PER_W, D_MODEL // 2), jnp.int32)]
        + [pltpu.VMEM((NUM_MOD, D_MODEL), jnp.float32)]
        + [pltpu.VMEM((ROWS, D_MODEL), jnp.float32)] * NBUF
        + [pltpu.SemaphoreType.DMA] * (2 * NBUF)
    ),
)
def _sc_kernel(x_hbm, mod_hbm, pe, o_hbm,
               pe_v, mod_v, b0, b1, b2, si0, si1, si2, so0, so1, so2):
    bufs = (b0, b1, b2)
    isems = (si0, si1, si2)
    osems = (so0, so1, so2)

    wid = lax.axis_index("s") * NC + lax.axis_index("c")
    base_s = wid * S_PER_W

    pltpu.sync_copy(pe.at[pl.ds(base_s, S_PER_W)], pe_v)
    pltpu.sync_copy(mod_hbm, mod_v)

    def chunk_idx(t):
        b, c = divmod(t, N_CHUNKS)
        return b, base_s + c * ROWS, c * ROWS

    def start_in(t):
        st = t % NBUF
        b, r0, _ = chunk_idx(t)
        pltpu.make_async_copy(
            x_hbm.at[b, pl.ds(r0, ROWS)], bufs[st], isems[st]).start()

    def wait_in(t):
        st = t % NBUF
        b, r0, _ = chunk_idx(t)
        pltpu.make_async_copy(
            x_hbm.at[b, pl.ds(r0, ROWS)], bufs[st], isems[st]).wait()

    def start_out(t):
        st = t % NBUF
        b, r0, _ = chunk_idx(t)
        pltpu.make_async_copy(
            bufs[st], o_hbm.at[b, pl.ds(r0, ROWS)], osems[st]).start()

    def wait_out(t):
        st = t % NBUF
        b, r0, _ = chunk_idx(t)
        pltpu.make_async_copy(
            bufs[st], o_hbm.at[b, pl.ds(r0, ROWS)], osems[st]).wait()

    start_in(0)

    for t in range(CH_PER_M):
        if t + 1 < CH_PER_M:
            if t - NBUF + 1 >= 0:
                wait_out(t - NBUF + 1)
            start_in(t + 1)
        wait_in(t)

        buf = bufs[t % NBUF]
        _, _, pr0 = chunk_idx(t)

        def gloop(g, _, buf=buf, pr0=pr0):
            mva = mod_v[0, pl.ds(g * 32, LANES)]
            mvb = mod_v[0, pl.ds(g * 32 + LANES, LANES)]

            @plsc.parallel_loop(0, ROWS, unroll=4)
            def rr(r):
                w = pe_v[pr0 + r, pl.ds(g * LANES, LANES)]
                a, b = plsc.unpack(
                    plsc.bitcast(w, jnp.bfloat16),
                    format=plsc.PackFormat.INTERLEAVED,
                    preferred_element_type=jnp.float32)
                plsc.addupdate(buf.at[r, pl.ds(g * 32, LANES)], a + mva)
                plsc.addupdate(buf.at[r, pl.ds(g * 32 + LANES, LANES)], b + mvb)

            return 0

        lax.fori_loop(0, D_MODEL // 32, gloop, 0)

        start_out(t)
    for t in range(CH_PER_M - NBUF, CH_PER_M):
        wait_out(t)


TC_BS = 256


def _tc_body(xi_ref, xn_ref, pe_ref, mod_ref, oi_ref, on_ref):
    pe_blk = pe_ref[...].astype(jnp.float32)
    oi_ref[...] = xi_ref[...] + (pe_blk + mod_ref[1][None, :])[None, :, :]
    on_ref[...] = xn_ref[...] + (pe_blk + mod_ref[2][None, :])[None, :, :]


def _tc_call(x_img, x_nlp, pe2d, mod_emb):
    return pl.pallas_call(
        _tc_body,
        grid=(S // TC_BS,),
        in_specs=[
            pl.BlockSpec((B, TC_BS, D_MODEL), lambda i: (0, i, 0)),
            pl.BlockSpec((B, TC_BS, D_MODEL), lambda i: (0, i, 0)),
            pl.BlockSpec((TC_BS, D_MODEL), lambda i: (i, 0)),
            pl.BlockSpec((NUM_MOD, D_MODEL), lambda i: (0, 0)),
        ],
        out_specs=[
            pl.BlockSpec((B, TC_BS, D_MODEL), lambda i: (0, i, 0)),
            pl.BlockSpec((B, TC_BS, D_MODEL), lambda i: (0, i, 0)),
        ],
        out_shape=[jax.ShapeDtypeStruct((B, S, D_MODEL), jnp.float32)] * 2,
        compiler_params=pltpu.CompilerParams(
            vmem_limit_bytes=100 * 1024 * 1024, skip_device_barrier=True),
    )(x_img, x_nlp, pe2d, mod_emb)


def kernel(x_global, x_img, x_nlp, mod_emb):
    out_g = _sc_kernel(x_global, mod_emb, _pe_packed_const())
    out_i, out_n = _tc_call(x_img, x_nlp, _pe_bf16_const(), mod_emb)
    return (out_g, out_i, out_n)


# speedup vs baseline: 1.0027x; 1.0027x over previous
import functools
import math

import jax
import jax.numpy as jnp
import numpy as np
from jax import lax
from jax.experimental import pallas as pl
from jax.experimental.pallas import tpu as pltpu
from jax.experimental.pallas import tpu_sc as plsc

D_MODEL = 1024
B = 4
S = 2048
NUM_MOD = 3

NC = 2
NS = 16
NW = NC * NS
S_PER_W = S // NW
ROWS = 16
N_CHUNKS = S_PER_W // ROWS
CH_PER_M = B * N_CHUNKS
LANES = 16
NBUF = 3


def _pe_np(d_model=D_MODEL, max_len=S):
    position = np.arange(max_len, dtype=np.float32)[:, None]
    div_term = np.exp(
        np.arange(0, d_model, 2, dtype=np.float32) * (-math.log(10000.0) / d_model))
    pe = np.zeros((max_len, d_model), dtype=np.float32)
    pe[:, 0::2] = np.sin(position * div_term)
    pe[:, 1::2] = np.cos(position * div_term)
    return pe


def _pe_bf16_const():
    import ml_dtypes
    return jnp.asarray(_pe_np().astype(ml_dtypes.bfloat16))


def _pe_packed_const():
    import ml_dtypes
    pe = _pe_np().reshape(S, D_MODEL // 32, 32)
    bits = pe.astype(ml_dtypes.bfloat16).view(np.uint16).astype(np.uint32)
    lo, hi = bits[:, :, :16], bits[:, :, 16:]
    words = (lo | (hi << 16)).astype(np.uint32).view(np.int32)
    return jnp.asarray(words.reshape(S, D_MODEL // 2))


_MESH = plsc.VectorSubcoreMesh(core_axis_name="c", subcore_axis_name="s")


@functools.partial(
    pl.kernel,
    mesh=_MESH,
    out_type=jax.ShapeDtypeStruct((B, S, D_MODEL), jnp.float32),
    compiler_params=pltpu.CompilerParams(
        use_tc_tiling_on_sc=True, needs_layout_passes=False),
    scratch_types=(
        [pltpu.VMEM((S_PER_W, D_MODEL // 2), jnp.int32)]
        + [pltpu.VMEM((NUM_MOD, D_MODEL), jnp.float32)]
        + [pltpu.VMEM((ROWS, D_MODEL), jnp.float32)] * NBUF
        + [pltpu.SemaphoreType.DMA] * (2 * NBUF)
    ),
)
def _sc_kernel(x_hbm, mod_hbm, pe, o_hbm,
               pe_v, mod_v, b0, b1, b2, si0, si1, si2, so0, so1, so2):
    bufs = (b0, b1, b2)
    isems = (si0, si1, si2)
    osems = (so0, so1, so2)

    wid = lax.axis_index("s") * NC + lax.axis_index("c")
    base_s = wid * S_PER_W

    pltpu.sync_copy(pe.at[pl.ds(base_s, S_PER_W)], pe_v)
    pltpu.sync_copy(mod_hbm, mod_v)

    def chunk_idx(t):
        b, c = divmod(t, N_CHUNKS)
        return b, base_s + c * ROWS, c * ROWS

    def start_in(t):
        st = t % NBUF
        b, r0, _ = chunk_idx(t)
        pltpu.make_async_copy(
            x_hbm.at[b, pl.ds(r0, ROWS)], bufs[st], isems[st]).start()

    def wait_in(t):
        st = t % NBUF
        b, r0, _ = chunk_idx(t)
        pltpu.make_async_copy(
            x_hbm.at[b, pl.ds(r0, ROWS)], bufs[st], isems[st]).wait()

    def start_out(t):
        st = t % NBUF
        b, r0, _ = chunk_idx(t)
        pltpu.make_async_copy(
            bufs[st], o_hbm.at[b, pl.ds(r0, ROWS)], osems[st]).start()

    def wait_out(t):
        st = t % NBUF
        b, r0, _ = chunk_idx(t)
        pltpu.make_async_copy(
            bufs[st], o_hbm.at[b, pl.ds(r0, ROWS)], osems[st]).wait()

    start_in(0)

    for t in range(CH_PER_M):
        if t + 1 < CH_PER_M:
            if t - NBUF + 1 >= 0:
                wait_out(t - NBUF + 1)
            start_in(t + 1)
        wait_in(t)

        buf = bufs[t % NBUF]
        _, _, pr0 = chunk_idx(t)

        def gloop(g, _, buf=buf, pr0=pr0):
            mva = mod_v[0, pl.ds(g * 32, LANES)]
            mvb = mod_v[0, pl.ds(g * 32 + LANES, LANES)]

            @plsc.parallel_loop(0, ROWS, unroll=4)
            def rr(r):
                w = pe_v[pr0 + r, pl.ds(g * LANES, LANES)]
                a, b = plsc.unpack(
                    plsc.bitcast(w, jnp.bfloat16),
                    format=plsc.PackFormat.INTERLEAVED,
                    preferred_element_type=jnp.float32)
                plsc.addupdate(buf.at[r, pl.ds(g * 32, LANES)], a + mva)
                plsc.addupdate(buf.at[r, pl.ds(g * 32 + LANES, LANES)], b + mvb)

            return 0

        lax.fori_loop(0, D_MODEL // 32, gloop, 0)

        start_out(t)
    for t in range(CH_PER_M - NBUF, CH_PER_M):
        wait_out(t)


TC_BS = 256


def _tc_body(xi_ref, xn_ref, pe_ref, mod_ref, oi_ref, on_ref):
    pe_blk = pe_ref[...].astype(jnp.float32)
    oi_ref[...] = xi_ref[...] + (pe_blk + mod_ref[1][None, :])[None, :, :]
    on_ref[...] = xn_ref[...] + (pe_blk + mod_ref[2][None, :])[None, :, :]


def _tc_call(x_img, x_nlp, pe2d, mod_emb):
    return pl.pallas_call(
        _tc_body,
        grid=(S // TC_BS,),
        in_specs=[
            pl.BlockSpec((B, TC_BS, D_MODEL), lambda i: (0, i, 0)),
            pl.BlockSpec((B, TC_BS, D_MODEL), lambda i: (0, i, 0)),
            pl.BlockSpec((TC_BS, D_MODEL), lambda i: (i, 0)),
            pl.BlockSpec((NUM_MOD, D_MODEL), lambda i: (0, 0)),
        ],
        out_specs=[
            pl.BlockSpec((B, TC_BS, D_MODEL), lambda i: (0, i, 0)),
            pl.BlockSpec((B, TC_BS, D_MODEL), lambda i: (0, i, 0)),
        ],
        out_shape=[jax.ShapeDtypeStruct((B, S, D_MODEL), jnp.float32)] * 2,
        compiler_params=pltpu.CompilerParams(vmem_limit_bytes=100 * 1024 * 1024),
    )(x_img, x_nlp, pe2d, mod_emb)


def kernel(x_global, x_img, x_nlp, mod_emb):
    out_g = _sc_kernel(x_global, mod_emb, _pe_packed_const())
    out_i, out_n = _tc_call(x_img, x_nlp, _pe_bf16_const(), mod_emb)
    return (out_g, out_i, out_n)
